# baseline (speedup 1.0000x reference)
import numpy as np
import jax
import jax.numpy as jnp
from jax.experimental import pallas as pl
from jax.experimental.pallas import tpu as pltpu

_GRID_SIZE = 5
_SPLINE_ORDER = 3
_GK = _GRID_SIZE + _SPLINE_ORDER
_NK = _GRID_SIZE + 2 * _SPLINE_ORDER + 1
_NI = _NK - 1

_KNOTS = [
    float(np.float32(t) * np.float32(2.0 / _GRID_SIZE) - np.float32(1.0))
    for t in range(-_SPLINE_ORDER, _GRID_SIZE + _SPLINE_ORDER + 1)
]
_K0 = _KNOTS[0]
_H = _KNOTS[1] - _KNOTS[0]
_INV_H = 1.0 / _H
_NEG_LOG2E = -1.4426950408889634

_BI = 64
_IC = 16


def _basis_piece_coeffs():
    K = np.array(_KNOTS, np.float64)
    ts = np.array([0.125, 0.375, 0.625, 0.875])
    T = np.zeros((_NI, _GK, 4))
    vand = np.vander(ts, 4, increasing=True)
    for m in range(_NI):
        xs = (K[m] + ts * (K[m + 1] - K[m]))[:, None]
        b = ((xs >= K[None, :-1]) & (xs < K[None, 1:])).astype(np.float64)
        for k in range(1, _SPLINE_ORDER + 1):
            left = (xs - K[None, :-(k + 1)]) / (K[None, k:-1] - K[None, :-(k + 1)]) * b[:, :-1]
            right = (K[None, k + 1:] - xs) / (K[None, k + 1:] - K[None, 1:-k]) * b[:, 1:]
            b = left + right
        T[m] = np.linalg.solve(vand, b).T
    return T


_PIECE_T = _basis_piece_coeffs()


def _bspline_bases(r):
    K = _KNOTS
    s = [jnp.where(r >= K[m], 1.0, 0.0).astype(r.dtype) for m in range(_NK)]
    d = [r - K[m] for m in range(_NK)]
    b = [s[m] - s[m + 1] for m in range(_NK - 1)]
    for k in range(1, _SPLINE_ORDER + 1):
        b = [
            d[m] * (b[m] * (1.0 / (K[m + k] - K[m])))
            - d[m + k + 1] * (b[m + 1] * (1.0 / (K[m + k + 1] - K[m + 1])))
            for m in range(len(b) - 1)
        ]
    return b


def _silu(v):
    return v * (1.0 / (1.0 + jnp.exp2(v * jnp.float32(_NEG_LOG2E))))


def _kan_mm(xx, wbT_ref, ws_ref):
    out = jnp.dot(_silu(xx), wbT_ref[...], preferred_element_type=jnp.float32)
    for g, bg in enumerate(_bspline_bases(xx)):
        out += jnp.dot(bg, ws_ref[g], preferred_element_type=jnp.float32)
    return out


def _tree_pick(masks, leaves, lo, hi):
    if lo == hi:
        return leaves[lo]
    mid = (lo + hi) // 2
    lo_t = _tree_pick(masks, leaves, lo, mid)
    hi_t = _tree_pick(masks, leaves, mid + 1, hi)
    return jnp.where(masks[mid], hi_t, lo_t)


def _fused_kernel(x_ref, xd_ref, x2_ref, adj_ref, fwb2_ref, sgn_ref, aco_ref,
                  mwbT_ref, mws_ref, uwbT_ref, uws_ref, out_ref):
    i = pl.program_id(1)
    x2full = x2_ref[0]
    fwb2 = fwb2_ref[0][None, None, :]
    sgn2 = sgn_ref[0][None, None, :]
    leaves = [[aco_ref[p * 13 + mi][None, :, :] for mi in range(13)]
              for p in range(2)]

    en_parts = []
    for ic in range(_BI // _IC):
        xi2 = xd_ref[0, pl.ds(i * _BI + ic * _IC, _IC), :]
        r = xi2[:, None, :] - x2full[None, :, :]
        t0 = r * jnp.float32(_INV_H) - jnp.float32(_K0 * _INV_H)
        mf = jnp.floor(t0)
        t = t0 - mf
        masks = {mid: mf >= jnp.float32(mid) for mid in range(12)}
        p32 = _tree_pick(masks, leaves[0], 0, 12)
        p10 = _tree_pick(masks, leaves[1], 0, 12)
        c3 = pltpu.bitcast(p32 & jnp.int32(-65536), jnp.float32)
        c2 = pltpu.bitcast(p32 << 16, jnp.float32)
        c1 = pltpu.bitcast(p10 & jnp.int32(-65536), jnp.float32)
        c0 = pltpu.bitcast(p10 << 16, jnp.float32)
        f = ((c3 * t + c2) * t + c1) * t + c0
        f += _silu(r) * fwb2
        s1 = jnp.sum(f, axis=-1)
        s2 = jnp.sum(f * sgn2, axis=-1)
        en_parts.append(jnp.concatenate([s1 + s2, s1 - s2], axis=-1))
    energy = jnp.concatenate(en_parts, axis=0)

    adjb = adj_ref[0]
    energy = jnp.where(adjb == 0, jnp.float32(-1e9), energy)
    emax = jnp.max(energy, axis=-1, keepdims=True)
    p = jnp.exp(energy - emax)
    alpha = p / jnp.sum(p, axis=-1, keepdims=True)

    msg = _kan_mm(x_ref[0], mwbT_ref, mws_ref)
    aggr = jnp.dot(alpha, msg, preferred_element_type=jnp.float32)

    xi_blk = x_ref[0, pl.ds(i * _BI, _BI), :]
    comb = jnp.concatenate([xi_blk, aggr], axis=-1)
    out_ref[0] = _kan_mm(comb, uwbT_ref, uws_ref)


def _pack_pair(hi, lo):
    hb = jax.lax.bitcast_convert_type(hi.astype(jnp.bfloat16), jnp.uint16)
    lb = jax.lax.bitcast_convert_type(lo.astype(jnp.bfloat16), jnp.uint16)
    packed = (hb.astype(jnp.uint32) << 16) | lb.astype(jnp.uint32)
    return jax.lax.bitcast_convert_type(packed, jnp.int32)


def kernel(x, adj, fw_base, fw_spline, fw_scaler, mw_base, mw_spline,
           mw_scaler, uw_base, uw_spline, uw_scaler):
    B, N, C = x.shape
    O = mw_base.shape[0]
    H = N // 2

    xd = jnp.tile(x, (1, 1, 2))
    x2 = jnp.concatenate([x[:, :H, :], x[:, H:, :]], axis=-1)
    fw = (fw_spline * fw_scaler[..., None])[0]
    fw2 = jnp.tile(fw, (2, 1))
    fwb2 = jnp.tile(fw_base, (1, 2)) * 0.5
    sgn2 = jnp.concatenate(
        [jnp.ones((1, C), jnp.float32), -jnp.ones((1, C), jnp.float32)], axis=1)
    A = jnp.einsum('mgd,cg->dmc',
                   jnp.asarray(_PIECE_T * 0.5, jnp.float32), fw2)
    Ap = jnp.pad(A, ((0, 0), (1, 1), (0, 0)))
    aco = jnp.concatenate(
        [_pack_pair(Ap[3], Ap[2]), _pack_pair(Ap[1], Ap[0])], axis=0)
    aco = jnp.broadcast_to(aco[:, None, :], (26, H, 2 * C))
    mws = (mw_spline * mw_scaler[..., None]).transpose(2, 1, 0)
    uws = (uw_spline * uw_scaler[..., None]).transpose(2, 1, 0)

    return pl.pallas_call(
        _fused_kernel,
        out_shape=jax.ShapeDtypeStruct((B, N, O), jnp.float32),
        grid=(B, N // _BI),
        in_specs=[
            pl.BlockSpec((1, N, C), lambda b, i: (b, 0, 0)),
            pl.BlockSpec((1, N, 2 * C), lambda b, i: (b, 0, 0)),
            pl.BlockSpec((1, H, 2 * C), lambda b, i: (b, 0, 0)),
            pl.BlockSpec((1, _BI, N), lambda b, i: (b, i, 0)),
            pl.BlockSpec((1, 2 * C), lambda b, i: (0, 0)),
            pl.BlockSpec((1, 2 * C), lambda b, i: (0, 0)),
            pl.BlockSpec((2 * 13, H, 2 * C), lambda b, i: (0, 0, 0)),
            pl.BlockSpec((C, O), lambda b, i: (0, 0)),
            pl.BlockSpec((_GK, C, O), lambda b, i: (0, 0, 0)),
            pl.BlockSpec((C + O, O), lambda b, i: (0, 0)),
            pl.BlockSpec((_GK, C + O, O), lambda b, i: (0, 0, 0)),
        ],
        out_specs=pl.BlockSpec((1, _BI, O), lambda b, i: (b, i, 0)),
        compiler_params=pltpu.CompilerParams(
            dimension_semantics=("parallel", "arbitrary"),
        ),
        name="kangatconv_fused",
    )(x, xd, x2, adj, fwb2, sgn2, aco, mw_base.T, mws, uw_base.T, uws)


# speedup vs baseline: 1.3834x; 1.3834x over previous
import numpy as np
import jax
import jax.numpy as jnp
from jax.experimental import pallas as pl
from jax.experimental.pallas import tpu as pltpu

_GRID_SIZE = 5
_SPLINE_ORDER = 3
_GK = _GRID_SIZE + _SPLINE_ORDER
_NK = _GRID_SIZE + 2 * _SPLINE_ORDER + 1
_NI = _NK - 1

_KNOTS = [
    float(np.float32(t) * np.float32(2.0 / _GRID_SIZE) - np.float32(1.0))
    for t in range(-_SPLINE_ORDER, _GRID_SIZE + _SPLINE_ORDER + 1)
]
_K0 = _KNOTS[0]
_H = _KNOTS[1] - _KNOTS[0]
_INV_H = 1.0 / _H
_NEG_LOG2E = -1.4426950408889634

_BI = 64
_IC = 8


def _basis_piece_coeffs():
    K = np.array(_KNOTS, np.float64)
    ts = np.array([0.125, 0.375, 0.625, 0.875])
    T = np.zeros((_NI, _GK, 4))
    vand = np.vander(ts, 4, increasing=True)
    for m in range(_NI):
        xs = (K[m] + ts * (K[m + 1] - K[m]))[:, None]
        b = ((xs >= K[None, :-1]) & (xs < K[None, 1:])).astype(np.float64)
        for k in range(1, _SPLINE_ORDER + 1):
            left = (xs - K[None, :-(k + 1)]) / (K[None, k:-1] - K[None, :-(k + 1)]) * b[:, :-1]
            right = (K[None, k + 1:] - xs) / (K[None, k + 1:] - K[None, 1:-k]) * b[:, 1:]
            b = left + right
        T[m] = np.linalg.solve(vand, b).T
    return T


_PIECE_T = _basis_piece_coeffs()


def _bspline_bases(r):
    K = _KNOTS
    s = [jnp.where(r >= K[m], 1.0, 0.0).astype(r.dtype) for m in range(_NK)]
    d = [r - K[m] for m in range(_NK)]
    b = [s[m] - s[m + 1] for m in range(_NK - 1)]
    for k in range(1, _SPLINE_ORDER + 1):
        b = [
            d[m] * (b[m] * (1.0 / (K[m + k] - K[m])))
            - d[m + k + 1] * (b[m + 1] * (1.0 / (K[m + k + 1] - K[m + 1])))
            for m in range(len(b) - 1)
        ]
    return b


def _silu(v):
    return v * (1.0 / (1.0 + jnp.exp2(v * jnp.float32(_NEG_LOG2E))))


def _kan_mm(xx, wbT_ref, ws_ref):
    out = jnp.dot(_silu(xx), wbT_ref[...], preferred_element_type=jnp.float32)
    for g, bg in enumerate(_bspline_bases(xx)):
        out += jnp.dot(bg, ws_ref[g], preferred_element_type=jnp.float32)
    return out


def _tree_pick(masks, leaves, lo, hi):
    if lo == hi:
        return leaves[lo]
    mid = (lo + hi) // 2
    lo_t = _tree_pick(masks, leaves, lo, mid)
    hi_t = _tree_pick(masks, leaves, mid + 1, hi)
    return jnp.where(masks[mid], hi_t, lo_t)


def _fused_kernel(x_ref, xd_ref, x2_ref, adj_ref, fwb2_ref, sgn_ref, aco_ref,
                  mwbT_ref, mws_ref, uwbT_ref, uws_ref, out_ref):
    i = pl.program_id(1)
    x2full = x2_ref[0]
    fwb2 = fwb2_ref[0][None, None, :]
    sgn2 = sgn_ref[0][None, None, :]
    leaves = [[aco_ref[p * 13 + mi][None, None, :] for mi in range(13)]
              for p in range(2)]

    en_parts = []
    for ic in range(_BI // _IC):
        xi2 = xd_ref[0, pl.ds(i * _BI + ic * _IC, _IC), :]
        r = xi2[:, None, :] - x2full[None, :, :]
        t0 = r * jnp.float32(_INV_H) - jnp.float32(_K0 * _INV_H)
        mf = jnp.floor(t0)
        t = t0 - mf
        masks = {mid: mf >= jnp.float32(mid) for mid in range(12)}
        p32 = _tree_pick(masks, leaves[0], 0, 12)
        p10 = _tree_pick(masks, leaves[1], 0, 12)
        c3 = pltpu.bitcast(p32 & jnp.int32(-65536), jnp.float32)
        c2 = pltpu.bitcast(p32 << 16, jnp.float32)
        c1 = pltpu.bitcast(p10 & jnp.int32(-65536), jnp.float32)
        c0 = pltpu.bitcast(p10 << 16, jnp.float32)
        f = ((c3 * t + c2) * t + c1) * t + c0
        f += _silu(r) * fwb2
        s1 = jnp.sum(f, axis=-1)
        s2 = jnp.sum(f * sgn2, axis=-1)
        en_parts.append(jnp.concatenate([s1 + s2, s1 - s2], axis=-1))
    energy = jnp.concatenate(en_parts, axis=0)

    adjb = adj_ref[0]
    energy = jnp.where(adjb == 0, jnp.float32(-1e9), energy)
    emax = jnp.max(energy, axis=-1, keepdims=True)
    p = jnp.exp(energy - emax)
    alpha = p / jnp.sum(p, axis=-1, keepdims=True)

    msg = _kan_mm(x_ref[0], mwbT_ref, mws_ref)
    aggr = jnp.dot(alpha, msg, preferred_element_type=jnp.float32)

    xi_blk = x_ref[0, pl.ds(i * _BI, _BI), :]
    comb = jnp.concatenate([xi_blk, aggr], axis=-1)
    out_ref[0] = _kan_mm(comb, uwbT_ref, uws_ref)


def _pack_pair(hi, lo):
    hb = jax.lax.bitcast_convert_type(hi.astype(jnp.bfloat16), jnp.uint16)
    lb = jax.lax.bitcast_convert_type(lo.astype(jnp.bfloat16), jnp.uint16)
    packed = (hb.astype(jnp.uint32) << 16) | lb.astype(jnp.uint32)
    return jax.lax.bitcast_convert_type(packed, jnp.int32)


def kernel(x, adj, fw_base, fw_spline, fw_scaler, mw_base, mw_spline,
           mw_scaler, uw_base, uw_spline, uw_scaler):
    B, N, C = x.shape
    O = mw_base.shape[0]
    H = N // 2

    xd = jnp.tile(x, (1, 1, 2))
    x2 = jnp.concatenate([x[:, :H, :], x[:, H:, :]], axis=-1)
    fw = (fw_spline * fw_scaler[..., None])[0]
    fw2 = jnp.tile(fw, (2, 1))
    fwb2 = jnp.tile(fw_base, (1, 2)) * 0.5
    sgn2 = jnp.concatenate(
        [jnp.ones((1, C), jnp.float32), -jnp.ones((1, C), jnp.float32)], axis=1)
    A = jnp.einsum('mgd,cg->dmc',
                   jnp.asarray(_PIECE_T * 0.5, jnp.float32), fw2)
    Ap = jnp.pad(A, ((0, 0), (1, 1), (0, 0)))
    aco = jnp.concatenate(
        [_pack_pair(Ap[3], Ap[2]), _pack_pair(Ap[1], Ap[0])], axis=0)
    mws = (mw_spline * mw_scaler[..., None]).transpose(2, 1, 0)
    uws = (uw_spline * uw_scaler[..., None]).transpose(2, 1, 0)

    return pl.pallas_call(
        _fused_kernel,
        out_shape=jax.ShapeDtypeStruct((B, N, O), jnp.float32),
        grid=(B, N // _BI),
        in_specs=[
            pl.BlockSpec((1, N, C), lambda b, i: (b, 0, 0)),
            pl.BlockSpec((1, N, 2 * C), lambda b, i: (b, 0, 0)),
            pl.BlockSpec((1, H, 2 * C), lambda b, i: (b, 0, 0)),
            pl.BlockSpec((1, _BI, N), lambda b, i: (b, i, 0)),
            pl.BlockSpec((1, 2 * C), lambda b, i: (0, 0)),
            pl.BlockSpec((1, 2 * C), lambda b, i: (0, 0)),
            pl.BlockSpec((2 * 13, 2 * C), lambda b, i: (0, 0)),
            pl.BlockSpec((C, O), lambda b, i: (0, 0)),
            pl.BlockSpec((_GK, C, O), lambda b, i: (0, 0, 0)),
            pl.BlockSpec((C + O, O), lambda b, i: (0, 0)),
            pl.BlockSpec((_GK, C + O, O), lambda b, i: (0, 0, 0)),
        ],
        out_specs=pl.BlockSpec((1, _BI, O), lambda b, i: (b, i, 0)),
        compiler_params=pltpu.CompilerParams(
            dimension_semantics=("parallel", "arbitrary"),
        ),
        name="kangatconv_fused",
    )(x, xd, x2, adj, fwb2, sgn2, aco, mw_base.T, mws, uw_base.T, uws)


# speedup vs baseline: 1.4095x; 1.0189x over previous
import numpy as np
import jax
import jax.numpy as jnp
from jax.experimental import pallas as pl
from jax.experimental.pallas import tpu as pltpu

_GRID_SIZE = 5
_SPLINE_ORDER = 3
_GK = _GRID_SIZE + _SPLINE_ORDER
_NK = _GRID_SIZE + 2 * _SPLINE_ORDER + 1
_NI = _NK - 1

_KNOTS = [
    float(np.float32(t) * np.float32(2.0 / _GRID_SIZE) - np.float32(1.0))
    for t in range(-_SPLINE_ORDER, _GRID_SIZE + _SPLINE_ORDER + 1)
]
_K0 = _KNOTS[0]
_H = _KNOTS[1] - _KNOTS[0]
_INV_H = 1.0 / _H
_NEG_LOG2E = -1.4426950408889634

_BI = 64
_IC = 32


def _basis_piece_coeffs():
    K = np.array(_KNOTS, np.float64)
    ts = np.array([0.125, 0.375, 0.625, 0.875])
    T = np.zeros((_NI, _GK, 4))
    vand = np.vander(ts, 4, increasing=True)
    for m in range(_NI):
        xs = (K[m] + ts * (K[m + 1] - K[m]))[:, None]
        b = ((xs >= K[None, :-1]) & (xs < K[None, 1:])).astype(np.float64)
        for k in range(1, _SPLINE_ORDER + 1):
            left = (xs - K[None, :-(k + 1)]) / (K[None, k:-1] - K[None, :-(k + 1)]) * b[:, :-1]
            right = (K[None, k + 1:] - xs) / (K[None, k + 1:] - K[None, 1:-k]) * b[:, 1:]
            b = left + right
        T[m] = np.linalg.solve(vand, b).T
    return T


_PIECE_T = _basis_piece_coeffs()


def _bspline_bases(r):
    K = _KNOTS
    s = [jnp.where(r >= K[m], 1.0, 0.0).astype(r.dtype) for m in range(_NK)]
    d = [r - K[m] for m in range(_NK)]
    b = [s[m] - s[m + 1] for m in range(_NK - 1)]
    for k in range(1, _SPLINE_ORDER + 1):
        b = [
            d[m] * (b[m] * (1.0 / (K[m + k] - K[m])))
            - d[m + k + 1] * (b[m + 1] * (1.0 / (K[m + k + 1] - K[m + 1])))
            for m in range(len(b) - 1)
        ]
    return b


def _silu(v):
    return v * (1.0 / (1.0 + jnp.exp2(v * jnp.float32(_NEG_LOG2E))))


def _kan_mm(xx, wbT_ref, ws_ref):
    out = jnp.dot(_silu(xx), wbT_ref[...], preferred_element_type=jnp.float32)
    for g, bg in enumerate(_bspline_bases(xx)):
        out += jnp.dot(bg, ws_ref[g], preferred_element_type=jnp.float32)
    return out


def _tree_pick(masks, leaves, lo, hi):
    if lo == hi:
        return leaves[lo]
    mid = (lo + hi) // 2
    lo_t = _tree_pick(masks, leaves, lo, mid)
    hi_t = _tree_pick(masks, leaves, mid + 1, hi)
    return jnp.where(masks[mid], hi_t, lo_t)


def _fused_kernel(x_ref, xd_ref, x2_ref, adj_ref, fwb2_ref, sgn_ref, aco_ref,
                  mwbT_ref, mws_ref, uwbT_ref, uws_ref, out_ref):
    i = pl.program_id(1)
    x2full = x2_ref[0]
    fwb2 = fwb2_ref[0][None, None, :]
    sgn2 = sgn_ref[0][None, None, :]
    leaves = [[aco_ref[p * 13 + mi][None, None, :] for mi in range(13)]
              for p in range(2)]

    en_parts = []
    for ic in range(_BI // _IC):
        xi2 = xd_ref[0, pl.ds(i * _BI + ic * _IC, _IC), :]
        r = xi2[:, None, :] - x2full[None, :, :]
        t0 = r * jnp.float32(_INV_H) - jnp.float32(_K0 * _INV_H)
        mf = jnp.floor(t0)
        t = t0 - mf
        masks = {mid: mf >= jnp.float32(mid) for mid in range(12)}
        p32 = _tree_pick(masks, leaves[0], 0, 12)
        p10 = _tree_pick(masks, leaves[1], 0, 12)
        c3 = pltpu.bitcast(p32 & jnp.int32(-65536), jnp.float32)
        c2 = pltpu.bitcast(p32 << 16, jnp.float32)
        c1 = pltpu.bitcast(p10 & jnp.int32(-65536), jnp.float32)
        c0 = pltpu.bitcast(p10 << 16, jnp.float32)
        f = ((c3 * t + c2) * t + c1) * t + c0
        f += _silu(r) * fwb2
        s1 = jnp.sum(f, axis=-1)
        s2 = jnp.sum(f * sgn2, axis=-1)
        en_parts.append(jnp.concatenate([s1 + s2, s1 - s2], axis=-1))
    energy = jnp.concatenate(en_parts, axis=0)

    adjb = adj_ref[0]
    energy = jnp.where(adjb == 0, jnp.float32(-1e9), energy)
    emax = jnp.max(energy, axis=-1, keepdims=True)
    p = jnp.exp(energy - emax)
    alpha = p / jnp.sum(p, axis=-1, keepdims=True)

    msg = _kan_mm(x_ref[0], mwbT_ref, mws_ref)
    aggr = jnp.dot(alpha, msg, preferred_element_type=jnp.float32)

    xi_blk = x_ref[0, pl.ds(i * _BI, _BI), :]
    comb = jnp.concatenate([xi_blk, aggr], axis=-1)
    out_ref[0] = _kan_mm(comb, uwbT_ref, uws_ref)


def _pack_pair(hi, lo):
    hb = jax.lax.bitcast_convert_type(hi.astype(jnp.bfloat16), jnp.uint16)
    lb = jax.lax.bitcast_convert_type(lo.astype(jnp.bfloat16), jnp.uint16)
    packed = (hb.astype(jnp.uint32) << 16) | lb.astype(jnp.uint32)
    return jax.lax.bitcast_convert_type(packed, jnp.int32)


def kernel(x, adj, fw_base, fw_spline, fw_scaler, mw_base, mw_spline,
           mw_scaler, uw_base, uw_spline, uw_scaler):
    B, N, C = x.shape
    O = mw_base.shape[0]
    H = N // 2

    xd = jnp.tile(x, (1, 1, 2))
    x2 = jnp.concatenate([x[:, :H, :], x[:, H:, :]], axis=-1)
    fw = (fw_spline * fw_scaler[..., None])[0]
    fw2 = jnp.tile(fw, (2, 1))
    fwb2 = jnp.tile(fw_base, (1, 2)) * 0.5
    sgn2 = jnp.concatenate(
        [jnp.ones((1, C), jnp.float32), -jnp.ones((1, C), jnp.float32)], axis=1)
    A = jnp.einsum('mgd,cg->dmc',
                   jnp.asarray(_PIECE_T * 0.5, jnp.float32), fw2)
    Ap = jnp.pad(A, ((0, 0), (1, 1), (0, 0)))
    aco = jnp.concatenate(
        [_pack_pair(Ap[3], Ap[2]), _pack_pair(Ap[1], Ap[0])], axis=0)
    mws = (mw_spline * mw_scaler[..., None]).transpose(2, 1, 0)
    uws = (uw_spline * uw_scaler[..., None]).transpose(2, 1, 0)

    return pl.pallas_call(
        _fused_kernel,
        out_shape=jax.ShapeDtypeStruct((B, N, O), jnp.float32),
        grid=(B, N // _BI),
        in_specs=[
            pl.BlockSpec((1, N, C), lambda b, i: (b, 0, 0)),
            pl.BlockSpec((1, N, 2 * C), lambda b, i: (b, 0, 0)),
            pl.BlockSpec((1, H, 2 * C), lambda b, i: (b, 0, 0)),
            pl.BlockSpec((1, _BI, N), lambda b, i: (b, i, 0)),
            pl.BlockSpec((1, 2 * C), lambda b, i: (0, 0)),
            pl.BlockSpec((1, 2 * C), lambda b, i: (0, 0)),
            pl.BlockSpec((2 * 13, 2 * C), lambda b, i: (0, 0)),
            pl.BlockSpec((C, O), lambda b, i: (0, 0)),
            pl.BlockSpec((_GK, C, O), lambda b, i: (0, 0, 0)),
            pl.BlockSpec((C + O, O), lambda b, i: (0, 0)),
            pl.BlockSpec((_GK, C + O, O), lambda b, i: (0, 0, 0)),
        ],
        out_specs=pl.BlockSpec((1, _BI, O), lambda b, i: (b, i, 0)),
        compiler_params=pltpu.CompilerParams(
            dimension_semantics=("parallel", "arbitrary"),
        ),
        name="kangatconv_fused",
    )(x, xd, x2, adj, fwb2, sgn2, aco, mw_base.T, mws, uw_base.T, uws)


# speedup vs baseline: 1.4128x; 1.0023x over previous
import numpy as np
import jax
import jax.numpy as jnp
from jax.experimental import pallas as pl
from jax.experimental.pallas import tpu as pltpu

_GRID_SIZE = 5
_SPLINE_ORDER = 3
_GK = _GRID_SIZE + _SPLINE_ORDER
_NK = _GRID_SIZE + 2 * _SPLINE_ORDER + 1
_NI = _NK - 1

_KNOTS = [
    float(np.float32(t) * np.float32(2.0 / _GRID_SIZE) - np.float32(1.0))
    for t in range(-_SPLINE_ORDER, _GRID_SIZE + _SPLINE_ORDER + 1)
]
_K0 = _KNOTS[0]
_H = _KNOTS[1] - _KNOTS[0]
_INV_H = 1.0 / _H
_NEG_LOG2E = -1.4426950408889634

_BI = 64
_IC = 64


def _basis_piece_coeffs():
    K = np.array(_KNOTS, np.float64)
    ts = np.array([0.125, 0.375, 0.625, 0.875])
    T = np.zeros((_NI, _GK, 4))
    vand = np.vander(ts, 4, increasing=True)
    for m in range(_NI):
        xs = (K[m] + ts * (K[m + 1] - K[m]))[:, None]
        b = ((xs >= K[None, :-1]) & (xs < K[None, 1:])).astype(np.float64)
        for k in range(1, _SPLINE_ORDER + 1):
            left = (xs - K[None, :-(k + 1)]) / (K[None, k:-1] - K[None, :-(k + 1)]) * b[:, :-1]
            right = (K[None, k + 1:] - xs) / (K[None, k + 1:] - K[None, 1:-k]) * b[:, 1:]
            b = left + right
        T[m] = np.linalg.solve(vand, b).T
    return T


_PIECE_T = _basis_piece_coeffs()


def _bspline_bases(r):
    K = _KNOTS
    s = [jnp.where(r >= K[m], 1.0, 0.0).astype(r.dtype) for m in range(_NK)]
    d = [r - K[m] for m in range(_NK)]
    b = [s[m] - s[m + 1] for m in range(_NK - 1)]
    for k in range(1, _SPLINE_ORDER + 1):
        b = [
            d[m] * (b[m] * (1.0 / (K[m + k] - K[m])))
            - d[m + k + 1] * (b[m + 1] * (1.0 / (K[m + k + 1] - K[m + 1])))
            for m in range(len(b) - 1)
        ]
    return b


def _silu(v):
    return v * (1.0 / (1.0 + jnp.exp2(v * jnp.float32(_NEG_LOG2E))))


def _kan_mm(xx, wbT_ref, ws_ref):
    out = jnp.dot(_silu(xx), wbT_ref[...], preferred_element_type=jnp.float32)
    for g, bg in enumerate(_bspline_bases(xx)):
        out += jnp.dot(bg, ws_ref[g], preferred_element_type=jnp.float32)
    return out


def _tree_pick(masks, leaves, lo, hi):
    if lo == hi:
        return leaves[lo]
    mid = (lo + hi) // 2
    lo_t = _tree_pick(masks, leaves, lo, mid)
    hi_t = _tree_pick(masks, leaves, mid + 1, hi)
    return jnp.where(masks[mid], hi_t, lo_t)


def _fused_kernel(x_ref, xd_ref, x2_ref, adj_ref, fwb2_ref, sgn_ref, aco_ref,
                  mwbT_ref, mws_ref, uwbT_ref, uws_ref, out_ref):
    i = pl.program_id(1)
    x2full = x2_ref[0]
    fwb2 = fwb2_ref[0][None, None, :]
    sgn2 = sgn_ref[0][None, None, :]
    leaves = [[aco_ref[p * 13 + mi][None, None, :] for mi in range(13)]
              for p in range(2)]

    en_parts = []
    for ic in range(_BI // _IC):
        xi2 = xd_ref[0, pl.ds(i * _BI + ic * _IC, _IC), :]
        r = xi2[:, None, :] - x2full[None, :, :]
        t0 = r * jnp.float32(_INV_H) - jnp.float32(_K0 * _INV_H)
        mf = jnp.floor(t0)
        t = t0 - mf
        masks = {mid: mf >= jnp.float32(mid) for mid in range(12)}
        p32 = _tree_pick(masks, leaves[0], 0, 12)
        p10 = _tree_pick(masks, leaves[1], 0, 12)
        c3 = pltpu.bitcast(p32 & jnp.int32(-65536), jnp.float32)
        c2 = pltpu.bitcast(p32 << 16, jnp.float32)
        c1 = pltpu.bitcast(p10 & jnp.int32(-65536), jnp.float32)
        c0 = pltpu.bitcast(p10 << 16, jnp.float32)
        f = ((c3 * t + c2) * t + c1) * t + c0
        f += _silu(r) * fwb2
        s1 = jnp.sum(f, axis=-1)
        s2 = jnp.sum(f * sgn2, axis=-1)
        en_parts.append(jnp.concatenate([s1 + s2, s1 - s2], axis=-1))
    energy = jnp.concatenate(en_parts, axis=0)

    adjb = adj_ref[0]
    energy = jnp.where(adjb == 0, jnp.float32(-1e9), energy)
    emax = jnp.max(energy, axis=-1, keepdims=True)
    p = jnp.exp(energy - emax)
    alpha = p / jnp.sum(p, axis=-1, keepdims=True)

    msg = _kan_mm(x_ref[0], mwbT_ref, mws_ref)
    aggr = jnp.dot(alpha, msg, preferred_element_type=jnp.float32)

    xi_blk = x_ref[0, pl.ds(i * _BI, _BI), :]
    comb = jnp.concatenate([xi_blk, aggr], axis=-1)
    out_ref[0] = _kan_mm(comb, uwbT_ref, uws_ref)


def _pack_pair(hi, lo):
    hb = jax.lax.bitcast_convert_type(hi.astype(jnp.bfloat16), jnp.uint16)
    lb = jax.lax.bitcast_convert_type(lo.astype(jnp.bfloat16), jnp.uint16)
    packed = (hb.astype(jnp.uint32) << 16) | lb.astype(jnp.uint32)
    return jax.lax.bitcast_convert_type(packed, jnp.int32)


def kernel(x, adj, fw_base, fw_spline, fw_scaler, mw_base, mw_spline,
           mw_scaler, uw_base, uw_spline, uw_scaler):
    B, N, C = x.shape
    O = mw_base.shape[0]
    H = N // 2

    xd = jnp.tile(x, (1, 1, 2))
    x2 = jnp.concatenate([x[:, :H, :], x[:, H:, :]], axis=-1)
    fw = (fw_spline * fw_scaler[..., None])[0]
    fw2 = jnp.tile(fw, (2, 1))
    fwb2 = jnp.tile(fw_base, (1, 2)) * 0.5
    sgn2 = jnp.concatenate(
        [jnp.ones((1, C), jnp.float32), -jnp.ones((1, C), jnp.float32)], axis=1)
    A = jnp.einsum('mgd,cg->dmc',
                   jnp.asarray(_PIECE_T * 0.5, jnp.float32), fw2)
    Ap = jnp.pad(A, ((0, 0), (1, 1), (0, 0)))
    aco = jnp.concatenate(
        [_pack_pair(Ap[3], Ap[2]), _pack_pair(Ap[1], Ap[0])], axis=0)
    mws = (mw_spline * mw_scaler[..., None]).transpose(2, 1, 0)
    uws = (uw_spline * uw_scaler[..., None]).transpose(2, 1, 0)

    return pl.pallas_call(
        _fused_kernel,
        out_shape=jax.ShapeDtypeStruct((B, N, O), jnp.float32),
        grid=(B, N // _BI),
        in_specs=[
            pl.BlockSpec((1, N, C), lambda b, i: (b, 0, 0)),
            pl.BlockSpec((1, N, 2 * C), lambda b, i: (b, 0, 0)),
            pl.BlockSpec((1, H, 2 * C), lambda b, i: (b, 0, 0)),
            pl.BlockSpec((1, _BI, N), lambda b, i: (b, i, 0)),
            pl.BlockSpec((1, 2 * C), lambda b, i: (0, 0)),
            pl.BlockSpec((1, 2 * C), lambda b, i: (0, 0)),
            pl.BlockSpec((2 * 13, 2 * C), lambda b, i: (0, 0)),
            pl.BlockSpec((C, O), lambda b, i: (0, 0)),
            pl.BlockSpec((_GK, C, O), lambda b, i: (0, 0, 0)),
            pl.BlockSpec((C + O, O), lambda b, i: (0, 0)),
            pl.BlockSpec((_GK, C + O, O), lambda b, i: (0, 0, 0)),
        ],
        out_specs=pl.BlockSpec((1, _BI, O), lambda b, i: (b, i, 0)),
        compiler_params=pltpu.CompilerParams(
            dimension_semantics=("parallel", "arbitrary"),
        ),
        name="kangatconv_fused",
    )(x, xd, x2, adj, fwb2, sgn2, aco, mw_base.T, mws, uw_base.T, uws)


# speedup vs baseline: 1.4726x; 1.0424x over previous
import numpy as np
import jax
import jax.numpy as jnp
from jax.experimental import pallas as pl
from jax.experimental.pallas import tpu as pltpu

_GRID_SIZE = 5
_SPLINE_ORDER = 3
_GK = _GRID_SIZE + _SPLINE_ORDER
_NK = _GRID_SIZE + 2 * _SPLINE_ORDER + 1
_NI = _NK - 1

_KNOTS = [
    float(np.float32(t) * np.float32(2.0 / _GRID_SIZE) - np.float32(1.0))
    for t in range(-_SPLINE_ORDER, _GRID_SIZE + _SPLINE_ORDER + 1)
]
_K0 = _KNOTS[0]
_H = _KNOTS[1] - _KNOTS[0]
_INV_H = 1.0 / _H
_NEG_LOG2E = -1.4426950408889634

_BI = 64
_IC = 64


def _basis_piece_coeffs():
    K = np.array(_KNOTS, np.float64)
    ts = np.array([0.125, 0.375, 0.625, 0.875])
    T = np.zeros((_NI, _GK, 4))
    vand = np.vander(ts, 4, increasing=True)
    for m in range(_NI):
        xs = (K[m] + ts * (K[m + 1] - K[m]))[:, None]
        b = ((xs >= K[None, :-1]) & (xs < K[None, 1:])).astype(np.float64)
        for k in range(1, _SPLINE_ORDER + 1):
            left = (xs - K[None, :-(k + 1)]) / (K[None, k:-1] - K[None, :-(k + 1)]) * b[:, :-1]
            right = (K[None, k + 1:] - xs) / (K[None, k + 1:] - K[None, 1:-k]) * b[:, 1:]
            b = left + right
        T[m] = np.linalg.solve(vand, b).T
    return T


_PIECE_T = _basis_piece_coeffs()


def _bspline_bases(r):
    K = _KNOTS
    s = [jnp.where(r >= K[m], 1.0, 0.0).astype(r.dtype) for m in range(_NK)]
    d = [r - K[m] for m in range(_NK)]
    b = [s[m] - s[m + 1] for m in range(_NK - 1)]
    for k in range(1, _SPLINE_ORDER + 1):
        b = [
            d[m] * (b[m] * (1.0 / (K[m + k] - K[m])))
            - d[m + k + 1] * (b[m + 1] * (1.0 / (K[m + k + 1] - K[m + 1])))
            for m in range(len(b) - 1)
        ]
    return b


def _silu(v):
    return v * (1.0 / (1.0 + jnp.exp2(v * jnp.float32(_NEG_LOG2E))))


def _kan_mm(xx, wbT_ref, ws_ref):
    out = jnp.dot(_silu(xx), wbT_ref[...], preferred_element_type=jnp.float32)
    for g, bg in enumerate(_bspline_bases(xx)):
        out += jnp.dot(bg, ws_ref[g], preferred_element_type=jnp.float32)
    return out


def _tree_pick(masks, leaves, lo, hi):
    if lo == hi:
        return leaves[lo]
    mid = (lo + hi) // 2
    lo_t = _tree_pick(masks, leaves, lo, mid)
    hi_t = _tree_pick(masks, leaves, mid + 1, hi)
    return jnp.where(masks[mid], hi_t, lo_t)


def _fused_kernel(x_ref, xd_ref, x2_ref, adj_ref, fwb2_ref, sgn_ref, aco_ref,
                  mwbT_ref, mws_ref, uwbT_ref, uws_ref, out_ref, msg_scr):
    i = pl.program_id(1)

    @pl.when(i == 0)
    def _():
        msg_scr[...] = _kan_mm(x_ref[0], mwbT_ref, mws_ref)

    x2full = x2_ref[0]
    fwb2 = fwb2_ref[0][None, None, :]
    sgn2 = sgn_ref[0][None, None, :]
    leaves = [[aco_ref[p * 13 + mi][None, None, :] for mi in range(13)]
              for p in range(2)]

    en_parts = []
    for ic in range(_BI // _IC):
        xi2 = xd_ref[0, pl.ds(i * _BI + ic * _IC, _IC), :]
        r = xi2[:, None, :] - x2full[None, :, :]
        t0 = r * jnp.float32(_INV_H) - jnp.float32(_K0 * _INV_H)
        mf = jnp.floor(t0)
        t = t0 - mf
        masks = {mid: mf >= jnp.float32(mid) for mid in range(12)}
        p32 = _tree_pick(masks, leaves[0], 0, 12)
        p10 = _tree_pick(masks, leaves[1], 0, 12)
        c3 = pltpu.bitcast(p32 & jnp.int32(-65536), jnp.float32)
        c2 = pltpu.bitcast(p32 << 16, jnp.float32)
        c1 = pltpu.bitcast(p10 & jnp.int32(-65536), jnp.float32)
        c0 = pltpu.bitcast(p10 << 16, jnp.float32)
        f = ((c3 * t + c2) * t + c1) * t + c0
        f += _silu(r) * fwb2
        s1 = jnp.sum(f, axis=-1)
        s2 = jnp.sum(f * sgn2, axis=-1)
        en_parts.append(jnp.concatenate([s1 + s2, s1 - s2], axis=-1))
    energy = jnp.concatenate(en_parts, axis=0)

    adjb = adj_ref[0]
    energy = jnp.where(adjb == 0, jnp.float32(-1e9), energy)
    emax = jnp.max(energy, axis=-1, keepdims=True)
    p = jnp.exp(energy - emax)
    alpha = p / jnp.sum(p, axis=-1, keepdims=True)

    aggr = jnp.dot(alpha, msg_scr[...], preferred_element_type=jnp.float32)

    xi_blk = x_ref[0, pl.ds(i * _BI, _BI), :]
    comb = jnp.concatenate([xi_blk, aggr], axis=-1)
    out_ref[0] = _kan_mm(comb, uwbT_ref, uws_ref)


def _pack_pair(hi, lo):
    hb = jax.lax.bitcast_convert_type(hi.astype(jnp.bfloat16), jnp.uint16)
    lb = jax.lax.bitcast_convert_type(lo.astype(jnp.bfloat16), jnp.uint16)
    packed = (hb.astype(jnp.uint32) << 16) | lb.astype(jnp.uint32)
    return jax.lax.bitcast_convert_type(packed, jnp.int32)


def kernel(x, adj, fw_base, fw_spline, fw_scaler, mw_base, mw_spline,
           mw_scaler, uw_base, uw_spline, uw_scaler):
    B, N, C = x.shape
    O = mw_base.shape[0]
    H = N // 2

    xd = jnp.tile(x, (1, 1, 2))
    x2 = jnp.concatenate([x[:, :H, :], x[:, H:, :]], axis=-1)
    fw = (fw_spline * fw_scaler[..., None])[0]
    fw2 = jnp.tile(fw, (2, 1))
    fwb2 = jnp.tile(fw_base, (1, 2)) * 0.5
    sgn2 = jnp.concatenate(
        [jnp.ones((1, C), jnp.float32), -jnp.ones((1, C), jnp.float32)], axis=1)
    A = jnp.einsum('mgd,cg->dmc',
                   jnp.asarray(_PIECE_T * 0.5, jnp.float32), fw2)
    Ap = jnp.pad(A, ((0, 0), (1, 1), (0, 0)))
    aco = jnp.concatenate(
        [_pack_pair(Ap[3], Ap[2]), _pack_pair(Ap[1], Ap[0])], axis=0)
    mws = (mw_spline * mw_scaler[..., None]).transpose(2, 1, 0)
    uws = (uw_spline * uw_scaler[..., None]).transpose(2, 1, 0)

    return pl.pallas_call(
        _fused_kernel,
        out_shape=jax.ShapeDtypeStruct((B, N, O), jnp.float32),
        grid=(B, N // _BI),
        in_specs=[
            pl.BlockSpec((1, N, C), lambda b, i: (b, 0, 0)),
            pl.BlockSpec((1, N, 2 * C), lambda b, i: (b, 0, 0)),
            pl.BlockSpec((1, H, 2 * C), lambda b, i: (b, 0, 0)),
            pl.BlockSpec((1, _BI, N), lambda b, i: (b, i, 0)),
            pl.BlockSpec((1, 2 * C), lambda b, i: (0, 0)),
            pl.BlockSpec((1, 2 * C), lambda b, i: (0, 0)),
            pl.BlockSpec((2 * 13, 2 * C), lambda b, i: (0, 0)),
            pl.BlockSpec((C, O), lambda b, i: (0, 0)),
            pl.BlockSpec((_GK, C, O), lambda b, i: (0, 0, 0)),
            pl.BlockSpec((C + O, O), lambda b, i: (0, 0)),
            pl.BlockSpec((_GK, C + O, O), lambda b, i: (0, 0, 0)),
        ],
        out_specs=pl.BlockSpec((1, _BI, O), lambda b, i: (b, i, 0)),
        scratch_shapes=[pltpu.VMEM((N, O), jnp.float32)],
        compiler_params=pltpu.CompilerParams(
            dimension_semantics=("parallel", "arbitrary"),
        ),
        name="kangatconv_fused",
    )(x, xd, x2, adj, fwb2, sgn2, aco, mw_base.T, mws, uw_base.T, uws)
